# baseline (speedup 1.0000x reference)
import functools

import jax
import jax.numpy as jnp
from jax.experimental import pallas as pl
from jax.experimental.pallas import tpu as pltpu

_BLK = 2048
_NCH = 4


def _pool_kernel(idx_ref, x_ref, wa_ref, v_ref, wo_ref, b_ref, out_ref,
                 s_ref, p_ref, e_ref, xs_ref, *, blk, n_seg, nblk):
    i = pl.program_id(0)
    ch = blk // _NCH
    d = xs_ref.shape[2]

    @pl.when(i == 0)
    def _init():
        s_ref[...] = jnp.zeros_like(s_ref)
        p_ref[...] = jnp.zeros_like(p_ref)
        e_ref[...] = jnp.zeros_like(e_ref)
        xs_ref[...] = jnp.zeros_like(xs_ref)

    lo0 = idx_ref[0]
    hin = idx_ref[n_seg]
    head_base = i * blk
    tail_base = head_base - blk
    ping = jax.lax.rem(i, 2)
    pong = jax.lax.rem(i + 1, 2)
    head_on = (i < nblk) & (head_base + blk > lo0) & (head_base < hin)
    tail_on = (i > 0) & (tail_base + blk > lo0) & (tail_base < hin)

    @pl.when(head_on | tail_on)
    def _main():
        for c in range(_NCH):
            xc = x_ref[pl.ds(c * ch, ch), :].astype(jnp.bfloat16)
            hc = jnp.tanh(
                jax.lax.dot_general(xc, wa_ref[...],
                                    (((1,), (0,)), ((), ())),
                                    preferred_element_type=jnp.float32))
            ec = jax.lax.dot_general(v_ref[...], hc,
                                     (((1,), (1,)), ((), ())),
                                     preferred_element_type=jnp.float32)
            we = jnp.exp(jnp.clip(ec, -60.0, 60.0))
            e_ref[pl.ds(ping, 1), :, pl.ds(c * ch, ch)] = we[None]
            xs_ref[pl.ds(ping, 1), pl.ds(c * ch, ch), :] = xc[None]

        expe = e_ref[pl.ds(pong, 1)].reshape(1, blk)
        pos = tail_base + jax.lax.broadcasted_iota(jnp.int32, (1, blk), 1)
        cnt = jnp.zeros((1, blk), jnp.int32)
        for j in range(n_seg + 1):
            cnt = cnt + (pos >= idx_ref[j]).astype(jnp.int32)
        sid = jnp.where(tail_on, cnt - 1, -2)
        row = jax.lax.broadcasted_iota(jnp.int32, (n_seg, blk), 0)
        w = jnp.where(row == sid, expe, 0.0)
        s_ref[...] = s_ref[...] + jnp.sum(w, axis=1, keepdims=True)
        wb = w.astype(jnp.bfloat16)
        acc = p_ref[...]
        for c in range(_NCH):
            xc = xs_ref[pl.ds(pong, 1), pl.ds(c * ch, ch), :].reshape(ch, d)
            acc = acc + jax.lax.dot_general(
                wb[:, c * ch:(c + 1) * ch], xc,
                (((1,), (0,)), ((), ())),
                preferred_element_type=jnp.float32)
        p_ref[...] = acc

    @pl.when(i == nblk)
    def _fin():
        pooled = p_ref[...] / s_ref[...]
        out_ref[...] = b_ref[...] + jax.lax.dot_general(
            pooled, wo_ref[...], (((1,), (0,)), ((), ())),
            preferred_element_type=jnp.float32,
            precision=jax.lax.Precision.HIGHEST)


def kernel(atom_features, index_list, W_att, v_att, W_out, b_out):
    tok, d_in = atom_features.shape
    d_out = W_out.shape[1]
    n_seg = index_list.shape[0] - 1
    blk = _BLK
    nblk = tok // blk

    idx = index_list.astype(jnp.int32)
    wa = W_att.astype(jnp.bfloat16)
    v2 = v_att.reshape(1, d_in).astype(jnp.float32)
    b2 = b_out.reshape(1, d_out).astype(jnp.float32)

    grid_spec = pltpu.PrefetchScalarGridSpec(
        num_scalar_prefetch=1,
        grid=(nblk + 1,),
        in_specs=[
            pl.BlockSpec((blk, d_in),
                         lambda i, idx_ref: (jnp.minimum(i, nblk - 1), 0)),
            pl.BlockSpec((d_in, d_in), lambda i, idx_ref: (0, 0)),
            pl.BlockSpec((1, d_in), lambda i, idx_ref: (0, 0)),
            pl.BlockSpec((d_in, d_out), lambda i, idx_ref: (0, 0)),
            pl.BlockSpec((1, d_out), lambda i, idx_ref: (0, 0)),
        ],
        out_specs=pl.BlockSpec((n_seg, d_out), lambda i, idx_ref: (0, 0)),
        scratch_shapes=[
            pltpu.VMEM((n_seg, 1), jnp.float32),
            pltpu.VMEM((n_seg, d_in), jnp.float32),
            pltpu.VMEM((2, 1, blk), jnp.float32),
            pltpu.VMEM((2, blk, d_in), jnp.bfloat16),
        ],
    )
    fn = pl.pallas_call(
        functools.partial(_pool_kernel, blk=blk, n_seg=n_seg, nblk=nblk),
        grid_spec=grid_spec,
        out_shape=jax.ShapeDtypeStruct((n_seg, d_out), jnp.float32),
    )
    return fn(idx, atom_features, wa, v2, W_out, b2)


# speedup vs baseline: 1.0010x; 1.0010x over previous
import functools

import jax
import jax.numpy as jnp
from jax.experimental import pallas as pl
from jax.experimental.pallas import tpu as pltpu

_BLK = 2048
_NCH = 4


def _head_tail(idx_ref, x_ref, wa_ref, v_ref, s_ref, p_ref,
               ew_ref, xw_ref, er_ref, xr_ref, tail_on, tail_base,
               *, blk, n_seg):
    ch = blk // _NCH
    d = xr_ref.shape[1]

    for c in range(_NCH):
        xc = x_ref[pl.ds(c * ch, ch), :].astype(jnp.bfloat16)
        hc = jnp.tanh(
            jax.lax.dot_general(xc, wa_ref[...],
                                (((1,), (0,)), ((), ())),
                                preferred_element_type=jnp.float32))
        ec = jax.lax.dot_general(v_ref[...], hc,
                                 (((1,), (1,)), ((), ())),
                                 preferred_element_type=jnp.float32)
        ew_ref[:, pl.ds(c * ch, ch)] = jnp.exp(jnp.clip(ec, -60.0, 60.0))
        xw_ref[pl.ds(c * ch, ch), :] = xc

    expe = er_ref[...]
    pos = tail_base + jax.lax.broadcasted_iota(jnp.int32, (1, blk), 1)
    cnt = jnp.zeros((1, blk), jnp.int32)
    for j in range(n_seg + 1):
        cnt = cnt + (pos >= idx_ref[j]).astype(jnp.int32)
    sid = jnp.where(tail_on, cnt - 1, -2)
    row = jax.lax.broadcasted_iota(jnp.int32, (n_seg, blk), 0)
    w = jnp.where(row == sid, expe, 0.0)
    s_ref[...] = s_ref[...] + jnp.sum(w, axis=1, keepdims=True)
    wb = w.astype(jnp.bfloat16)
    acc = p_ref[...]
    for c in range(_NCH):
        acc = acc + jax.lax.dot_general(
            wb[:, c * ch:(c + 1) * ch], xr_ref[pl.ds(c * ch, ch), :],
            (((1,), (0,)), ((), ())),
            preferred_element_type=jnp.float32)
    p_ref[...] = acc


def _pool_kernel(idx_ref, x_ref, wa_ref, v_ref, wo_ref, b_ref, out_ref,
                 s_ref, p_ref, e0_ref, e1_ref, x0_ref, x1_ref,
                 *, blk, n_seg, nblk):
    i = pl.program_id(0)

    @pl.when(i == 0)
    def _init():
        s_ref[...] = jnp.zeros_like(s_ref)
        p_ref[...] = jnp.zeros_like(p_ref)
        e0_ref[...] = jnp.zeros_like(e0_ref)
        e1_ref[...] = jnp.zeros_like(e1_ref)
        x0_ref[...] = jnp.zeros_like(x0_ref)
        x1_ref[...] = jnp.zeros_like(x1_ref)

    lo0 = idx_ref[0]
    hin = idx_ref[n_seg]
    head_base = i * blk
    tail_base = head_base - blk
    head_on = (i < nblk) & (head_base + blk > lo0) & (head_base < hin)
    tail_on = (i > 0) & (tail_base + blk > lo0) & (tail_base < hin)
    run = head_on | tail_on
    even = jax.lax.rem(i, 2) == 0
    body = functools.partial(_head_tail, idx_ref, x_ref, wa_ref, v_ref,
                             s_ref, p_ref, blk=blk, n_seg=n_seg)

    @pl.when(run & even)
    def _main_even():
        body(e0_ref, x0_ref, e1_ref, x1_ref, tail_on, tail_base)

    @pl.when(run & jnp.logical_not(even))
    def _main_odd():
        body(e1_ref, x1_ref, e0_ref, x0_ref, tail_on, tail_base)

    @pl.when(i == nblk)
    def _fin():
        pooled = p_ref[...] / s_ref[...]
        out_ref[...] = b_ref[...] + jax.lax.dot_general(
            pooled, wo_ref[...], (((1,), (0,)), ((), ())),
            preferred_element_type=jnp.float32,
            precision=jax.lax.Precision.HIGHEST)


def kernel(atom_features, index_list, W_att, v_att, W_out, b_out):
    tok, d_in = atom_features.shape
    d_out = W_out.shape[1]
    n_seg = index_list.shape[0] - 1
    blk = _BLK
    nblk = tok // blk

    idx = index_list.astype(jnp.int32)
    wa = W_att.astype(jnp.bfloat16)
    v2 = v_att.reshape(1, d_in).astype(jnp.float32)
    b2 = b_out.reshape(1, d_out).astype(jnp.float32)

    grid_spec = pltpu.PrefetchScalarGridSpec(
        num_scalar_prefetch=1,
        grid=(nblk + 1,),
        in_specs=[
            pl.BlockSpec((blk, d_in),
                         lambda i, idx_ref: (jnp.minimum(i, nblk - 1), 0)),
            pl.BlockSpec((d_in, d_in), lambda i, idx_ref: (0, 0)),
            pl.BlockSpec((1, d_in), lambda i, idx_ref: (0, 0)),
            pl.BlockSpec((d_in, d_out), lambda i, idx_ref: (0, 0)),
            pl.BlockSpec((1, d_out), lambda i, idx_ref: (0, 0)),
        ],
        out_specs=pl.BlockSpec((n_seg, d_out), lambda i, idx_ref: (0, 0)),
        scratch_shapes=[
            pltpu.VMEM((n_seg, 1), jnp.float32),
            pltpu.VMEM((n_seg, d_in), jnp.float32),
            pltpu.VMEM((1, blk), jnp.float32),
            pltpu.VMEM((1, blk), jnp.float32),
            pltpu.VMEM((blk, d_in), jnp.bfloat16),
            pltpu.VMEM((blk, d_in), jnp.bfloat16),
        ],
    )
    fn = pl.pallas_call(
        functools.partial(_pool_kernel, blk=blk, n_seg=n_seg, nblk=nblk),
        grid_spec=grid_spec,
        out_shape=jax.ShapeDtypeStruct((n_seg, d_out), jnp.float32),
    )
    return fn(idx, atom_features, wa, v2, W_out, b2)


# speedup vs baseline: 1.1011x; 1.1000x over previous
import functools

import jax
import jax.numpy as jnp
from jax.experimental import pallas as pl
from jax.experimental.pallas import tpu as pltpu

_BLK = 2048


def _pool_kernel(idx_ref, x_ref, wa_ref, v_ref, wo_ref, b_ref, out_ref,
                 m_ref, s_ref, p_ref, *, blk, n_seg):
    i = pl.program_id(0)
    nblk = pl.num_programs(0)
    neg_inf = jnp.float32(-jnp.inf)

    @pl.when(i == 0)
    def _init():
        m_ref[...] = jnp.full_like(m_ref, neg_inf)
        s_ref[...] = jnp.zeros_like(s_ref)
        p_ref[...] = jnp.zeros_like(p_ref)

    base = i * blk
    lo0 = idx_ref[0]
    hin = idx_ref[n_seg]

    @pl.when((base + blk > lo0) & (base < hin))
    def _work():
        xb = x_ref[...]
        h = jnp.tanh(
            jax.lax.dot_general(xb, wa_ref[...],
                                (((1,), (0,)), ((), ())),
                                preferred_element_type=jnp.float32))
        e = jax.lax.dot_general(v_ref[...], h,
                                (((1,), (1,)), ((), ())),
                                preferred_element_type=jnp.float32)

        pos = base + jax.lax.broadcasted_iota(jnp.int32, (1, blk), 1)
        cnt = jnp.zeros((1, blk), jnp.int32)
        for j in range(n_seg + 1):
            cnt = cnt + (pos >= idx_ref[j]).astype(jnp.int32)
        row = jax.lax.broadcasted_iota(jnp.int32, (n_seg, blk), 0)
        mask = row == (cnt - 1)

        e_m = jnp.where(mask, e, neg_inf)
        bm = jnp.max(e_m, axis=1, keepdims=True)
        m_old = m_ref[...]
        m_new = jnp.maximum(m_old, bm)
        scale = jnp.where(jnp.isfinite(m_old),
                          jnp.exp(m_old - m_new), 0.0)
        w = jnp.where(mask, jnp.exp(e_m - m_new), 0.0)
        s_ref[...] = s_ref[...] * scale + jnp.sum(w, axis=1, keepdims=True)
        p_ref[...] = p_ref[...] * scale + jax.lax.dot_general(
            w, xb, (((1,), (0,)), ((), ())),
            preferred_element_type=jnp.float32)
        m_ref[...] = m_new

    @pl.when(i == nblk - 1)
    def _fin():
        pooled = p_ref[...] / s_ref[...]
        out_ref[...] = b_ref[...] + jax.lax.dot_general(
            pooled, wo_ref[...], (((1,), (0,)), ((), ())),
            preferred_element_type=jnp.float32,
            precision=jax.lax.Precision.HIGHEST)


def kernel(atom_features, index_list, W_att, v_att, W_out, b_out):
    tok, d_in = atom_features.shape
    d_out = W_out.shape[1]
    n_seg = index_list.shape[0] - 1
    blk = _BLK
    nblk = tok // blk

    idx = index_list.astype(jnp.int32)
    v2 = v_att.reshape(1, d_in).astype(jnp.float32)
    b2 = b_out.reshape(1, d_out).astype(jnp.float32)

    def _x_map(i, idx_ref):
        return (jnp.clip(i, idx_ref[0] // blk, (idx_ref[n_seg] - 1) // blk),
                0)

    grid_spec = pltpu.PrefetchScalarGridSpec(
        num_scalar_prefetch=1,
        grid=(nblk,),
        in_specs=[
            pl.BlockSpec((blk, d_in), _x_map),
            pl.BlockSpec((d_in, d_in), lambda i, idx_ref: (0, 0)),
            pl.BlockSpec((1, d_in), lambda i, idx_ref: (0, 0)),
            pl.BlockSpec((d_in, d_out), lambda i, idx_ref: (0, 0)),
            pl.BlockSpec((1, d_out), lambda i, idx_ref: (0, 0)),
        ],
        out_specs=pl.BlockSpec((n_seg, d_out), lambda i, idx_ref: (0, 0)),
        scratch_shapes=[
            pltpu.VMEM((n_seg, 1), jnp.float32),
            pltpu.VMEM((n_seg, 1), jnp.float32),
            pltpu.VMEM((n_seg, d_in), jnp.float32),
        ],
    )
    fn = pl.pallas_call(
        functools.partial(_pool_kernel, blk=blk, n_seg=n_seg),
        grid_spec=grid_spec,
        out_shape=jax.ShapeDtypeStruct((n_seg, d_out), jnp.float32),
    )
    return fn(idx, atom_features, W_att, v2, W_out, b2)
